# baseline (speedup 1.0000x reference)
import functools

import jax
import jax.numpy as jnp
from jax import lax
from jax.experimental import pallas as pl
from jax.experimental.pallas import tpu as pltpu
from jax.experimental.pallas import tpu_sc as plsc

N = 10000
D = 128
E = 320000
NC = 2
NS = 16
CH = 128
TCH = 2560
CPF = 160
CPS = 0
PC = 16
PAIRS = PC // 2
EP = TCH * CH
NP = 10240
SPT = NP // NS
R = 1000


def _sc_body(with_cnt, *refs):
    if with_cnt:
        (x_hbm, src_hbm, dst_hbm, agg_out, cnt_out,
         agg_sh, sidx0, sidx1, didx0, didx1, rows0, rows1,
         gsem0, gsem1, isem, cnt_sh, ones_v) = refs
    else:
        (x_hbm, src_hbm, dst_hbm, agg_out,
         agg_sh, sidx0, sidx1, didx0, didx1, rows0, rows1,
         gsem0, gsem1, isem) = refs
    c = lax.axis_index("c")
    s = lax.axis_index("s")
    row0 = s * SPT

    zv = jnp.zeros((16,), jnp.float32)

    def _zb(i, carry):
        rows0[i // 8, pl.ds((i % 8) * 16, 16)] = zv
        return carry

    lax.fori_loop(0, CH * 8, _zb, 0)
    for k in range(SPT // CH):
        pltpu.sync_copy(rows0, agg_sh.at[pl.ds(row0 + k * CH, CH), :])
    if with_cnt:
        ov = jnp.ones((16,), jnp.float32)
        for k in range(CH // 16):
            ones_v[pl.ds(k * 16, 16)] = ov
        for k in range(SPT // CH):
            pltpu.sync_copy(rows0.at[0], cnt_sh.at[pl.ds(row0 + k * CH, CH)])

    def _pipeline(qbase, npieces):
        pltpu.sync_copy(src_hbm.at[pl.ds(qbase, PC), :], sidx0)
        pltpu.sync_copy(dst_hbm.at[pl.ds(qbase, PC), :], didx0)
        for p in range(npieces):
            sib, dib = (sidx0, didx0) if p % 2 == 0 else (sidx1, didx1)
            if p < npieces - 1:
                sib_n, dib_n = (sidx1, didx1) if p % 2 == 0 else (sidx0, didx0)
                ip = pltpu.async_copy(
                    src_hbm.at[pl.ds(qbase + (p + 1) * PC, PC), :], sib_n,
                    isem)
                ip2 = pltpu.async_copy(
                    dst_hbm.at[pl.ds(qbase + (p + 1) * PC, PC), :], dib_n,
                    isem)
            pltpu.async_copy(x_hbm.at[sib.at[0]], rows0, gsem0)

            def _pair(i, carry):
                j0 = 2 * i
                j1 = j0 + 1
                pltpu.async_copy(x_hbm.at[sib.at[j1]], rows1, gsem1)
                pltpu.make_async_copy(
                    x_hbm.at[sib.at[j0]], rows0, gsem0).wait()
                pltpu.sync_copy(rows0, agg_sh.at[dib.at[j0]], add=True)
                if with_cnt:
                    pltpu.sync_copy(ones_v, cnt_sh.at[dib.at[j0]], add=True)

                @pl.when(i < PAIRS - 1)
                def _():
                    pltpu.async_copy(x_hbm.at[sib.at[j0 + 2]], rows0, gsem0)

                pltpu.make_async_copy(
                    x_hbm.at[sib.at[j1]], rows1, gsem1).wait()
                pltpu.sync_copy(rows1, agg_sh.at[dib.at[j1]], add=True)
                if with_cnt:
                    pltpu.sync_copy(ones_v, cnt_sh.at[dib.at[j1]], add=True)
                return carry

            lax.fori_loop(0, PAIRS, _pair, 0)
            if p < npieces - 1:
                ip.wait()
                ip2.wait()

    plsc.subcore_barrier()

    @pl.when(c == 0)
    def _():
        _pipeline(s * CPF, CPF // PC)

    if CPS > 0:
        @pl.when(c == 1)
        def _():
            _pipeline(NS * CPF + s * CPS, CPS // PC)

    plsc.subcore_barrier()

    for k in range(SPT // CH):
        pltpu.sync_copy(agg_sh.at[pl.ds(row0 + k * CH, CH), :],
                        agg_out.at[c, pl.ds(row0 + k * CH, CH), :])
    if with_cnt:
        pltpu.sync_copy(cnt_sh.at[pl.ds(row0, SPT)],
                        cnt_out.at[c, pl.ds(row0, SPT)])


@functools.cache
def _make_sc(with_cnt):
    mesh = plsc.VectorSubcoreMesh(core_axis_name="c", subcore_axis_name="s",
                                  num_cores=NC, num_subcores=NS)
    out_type = [jax.ShapeDtypeStruct((NC, NP, D), jnp.float32)]
    scratch = [
        pltpu.VMEM_SHARED((NP, D), jnp.float32),
        pltpu.VMEM((PC, CH), jnp.int32),
        pltpu.VMEM((PC, CH), jnp.int32),
        pltpu.VMEM((PC, CH), jnp.int32),
        pltpu.VMEM((PC, CH), jnp.int32),
        pltpu.VMEM((CH, D), jnp.float32),
        pltpu.VMEM((CH, D), jnp.float32),
        pltpu.SemaphoreType.DMA,
        pltpu.SemaphoreType.DMA,
        pltpu.SemaphoreType.DMA,
    ]
    if with_cnt:
        out_type.append(jax.ShapeDtypeStruct((NC, NP), jnp.float32))
        scratch += [
            pltpu.VMEM_SHARED((NP,), jnp.float32),
            pltpu.VMEM((CH,), jnp.float32),
        ]
    return pl.kernel(
        functools.partial(_sc_body, with_cnt),
        out_type=out_type,
        mesh=mesh,
        scratch_types=scratch,
    )


def _tc_body(relu, agg_ref, cnt_ref, xin_ref, wl_ref, wr_ref, b_ref, out_ref):
    cnt = cnt_ref[0] + cnt_ref[1]
    rec = 1.0 / jnp.maximum(cnt, 1.0)
    mean = (agg_ref[0] + agg_ref[1]) * rec
    acc = jnp.dot(mean, wl_ref[...], preferred_element_type=jnp.float32)
    acc = acc + jnp.dot(xin_ref[...], wr_ref[...],
                        preferred_element_type=jnp.float32)
    acc = acc + b_ref[...]
    out_ref[...] = jnp.maximum(acc, 0.0) if relu else acc


def _make_tc(relu):
    return pl.pallas_call(
        functools.partial(_tc_body, relu),
        grid=(N // R,),
        in_specs=[
            pl.BlockSpec((NC, R, D), lambda r: (0, r, 0)),
            pl.BlockSpec((NC, R, 1), lambda r: (0, r, 0)),
            pl.BlockSpec((R, D), lambda r: (r, 0)),
            pl.BlockSpec((D, D), lambda r: (0, 0)),
            pl.BlockSpec((D, D), lambda r: (0, 0)),
            pl.BlockSpec((1, D), lambda r: (0, 0)),
        ],
        out_specs=pl.BlockSpec((R, D), lambda r: (r, 0)),
        out_shape=jax.ShapeDtypeStruct((N, D), jnp.float32),
    )


_TC_RELU = _make_tc(True)
_TC_LIN = _make_tc(False)


def kernel(x, edge_index, W1_l, W1_r, b1, W2_l, W2_r, b2):
    pad = EP - E
    src_p = jnp.concatenate(
        [edge_index[0], jnp.zeros((pad,), jnp.int32)]).reshape(TCH, CH)
    dst_pad = N + (jnp.arange(pad, dtype=jnp.int32) % (NP - N))
    dst_p = jnp.concatenate([edge_index[1], dst_pad]).reshape(TCH, CH)

    agg1, cnt1 = _make_sc(True)(x, src_p, dst_p)
    cnt3 = cnt1.reshape(NC, NP, 1)
    h = _TC_RELU(agg1, cnt3, x, W1_l, W1_r, b1.reshape(1, D))
    agg2, = _make_sc(False)(h, src_p, dst_p)
    return _TC_LIN(agg2, cnt3, h, W2_l, W2_r, b2.reshape(1, D))


# speedup vs baseline: 1.1986x; 1.1986x over previous
import functools

import jax
import jax.numpy as jnp
from jax import lax
from jax.experimental import pallas as pl
from jax.experimental.pallas import tpu as pltpu
from jax.experimental.pallas import tpu_sc as plsc

N = 10000
D = 128
E = 320000
NC = 2
NS = 16
CH = 128
TCH = 2560
CPF = 112
CPS = 48
PC = 16
PAIRS = PC // 2
EP = TCH * CH
NP = 10240
SPT = NP // NS
R = 1000


def _sc_body(with_cnt, *refs):
    if with_cnt:
        (x_hbm, src_hbm, dst_hbm, agg_out, cnt_out,
         agg_sh, sidx0, sidx1, didx0, didx1, rows0, rows1,
         gsem0, gsem1, isem, cnt_sh, ones_v) = refs
    else:
        (x_hbm, src_hbm, dst_hbm, agg_out,
         agg_sh, sidx0, sidx1, didx0, didx1, rows0, rows1,
         gsem0, gsem1, isem) = refs
    c = lax.axis_index("c")
    s = lax.axis_index("s")
    row0 = s * SPT

    zv = jnp.zeros((16,), jnp.float32)

    def _zb(i, carry):
        rows0[i // 8, pl.ds((i % 8) * 16, 16)] = zv
        return carry

    lax.fori_loop(0, CH * 8, _zb, 0)
    for k in range(SPT // CH):
        pltpu.sync_copy(rows0, agg_sh.at[pl.ds(row0 + k * CH, CH), :])
    if with_cnt:
        ov = jnp.ones((16,), jnp.float32)
        for k in range(CH // 16):
            ones_v[pl.ds(k * 16, 16)] = ov
        for k in range(SPT // CH):
            pltpu.sync_copy(rows0.at[0], cnt_sh.at[pl.ds(row0 + k * CH, CH)])

    def _pipeline(qbase, npieces):
        pltpu.sync_copy(src_hbm.at[pl.ds(qbase, PC), :], sidx0)
        pltpu.sync_copy(dst_hbm.at[pl.ds(qbase, PC), :], didx0)
        for p in range(npieces):
            sib, dib = (sidx0, didx0) if p % 2 == 0 else (sidx1, didx1)
            if p < npieces - 1:
                sib_n, dib_n = (sidx1, didx1) if p % 2 == 0 else (sidx0, didx0)
                ip = pltpu.async_copy(
                    src_hbm.at[pl.ds(qbase + (p + 1) * PC, PC), :], sib_n,
                    isem)
                ip2 = pltpu.async_copy(
                    dst_hbm.at[pl.ds(qbase + (p + 1) * PC, PC), :], dib_n,
                    isem)
            pltpu.async_copy(x_hbm.at[sib.at[0]], rows0, gsem0)

            def _pair(i, carry):
                j0 = 2 * i
                j1 = j0 + 1
                pltpu.async_copy(x_hbm.at[sib.at[j1]], rows1, gsem1)
                pltpu.make_async_copy(
                    x_hbm.at[sib.at[j0]], rows0, gsem0).wait()
                pltpu.sync_copy(rows0, agg_sh.at[dib.at[j0]], add=True)
                if with_cnt:
                    pltpu.sync_copy(ones_v, cnt_sh.at[dib.at[j0]], add=True)

                @pl.when(i < PAIRS - 1)
                def _():
                    pltpu.async_copy(x_hbm.at[sib.at[j0 + 2]], rows0, gsem0)

                pltpu.make_async_copy(
                    x_hbm.at[sib.at[j1]], rows1, gsem1).wait()
                pltpu.sync_copy(rows1, agg_sh.at[dib.at[j1]], add=True)
                if with_cnt:
                    pltpu.sync_copy(ones_v, cnt_sh.at[dib.at[j1]], add=True)
                return carry

            lax.fori_loop(0, PAIRS, _pair, 0)
            if p < npieces - 1:
                ip.wait()
                ip2.wait()

    plsc.subcore_barrier()

    @pl.when(c == 0)
    def _():
        _pipeline(s * CPF, CPF // PC)

    if CPS > 0:
        @pl.when(c == 1)
        def _():
            _pipeline(NS * CPF + s * CPS, CPS // PC)

    plsc.subcore_barrier()

    for k in range(SPT // CH):
        pltpu.sync_copy(agg_sh.at[pl.ds(row0 + k * CH, CH), :],
                        agg_out.at[c, pl.ds(row0 + k * CH, CH), :])
    if with_cnt:
        pltpu.sync_copy(cnt_sh.at[pl.ds(row0, SPT)],
                        cnt_out.at[c, pl.ds(row0, SPT)])


@functools.cache
def _make_sc(with_cnt):
    mesh = plsc.VectorSubcoreMesh(core_axis_name="c", subcore_axis_name="s",
                                  num_cores=NC, num_subcores=NS)
    out_type = [jax.ShapeDtypeStruct((NC, NP, D), jnp.float32)]
    scratch = [
        pltpu.VMEM_SHARED((NP, D), jnp.float32),
        pltpu.VMEM((PC, CH), jnp.int32),
        pltpu.VMEM((PC, CH), jnp.int32),
        pltpu.VMEM((PC, CH), jnp.int32),
        pltpu.VMEM((PC, CH), jnp.int32),
        pltpu.VMEM((CH, D), jnp.float32),
        pltpu.VMEM((CH, D), jnp.float32),
        pltpu.SemaphoreType.DMA,
        pltpu.SemaphoreType.DMA,
        pltpu.SemaphoreType.DMA,
    ]
    if with_cnt:
        out_type.append(jax.ShapeDtypeStruct((NC, NP), jnp.float32))
        scratch += [
            pltpu.VMEM_SHARED((NP,), jnp.float32),
            pltpu.VMEM((CH,), jnp.float32),
        ]
    return pl.kernel(
        functools.partial(_sc_body, with_cnt),
        out_type=out_type,
        mesh=mesh,
        scratch_types=scratch,
    )


def _tc_body(relu, agg_ref, cnt_ref, xin_ref, wl_ref, wr_ref, b_ref, out_ref):
    cnt = cnt_ref[0] + cnt_ref[1]
    rec = 1.0 / jnp.maximum(cnt, 1.0)
    mean = (agg_ref[0] + agg_ref[1]) * rec
    acc = jnp.dot(mean, wl_ref[...], preferred_element_type=jnp.float32)
    acc = acc + jnp.dot(xin_ref[...], wr_ref[...],
                        preferred_element_type=jnp.float32)
    acc = acc + b_ref[...]
    out_ref[...] = jnp.maximum(acc, 0.0) if relu else acc


def _make_tc(relu):
    return pl.pallas_call(
        functools.partial(_tc_body, relu),
        grid=(N // R,),
        in_specs=[
            pl.BlockSpec((NC, R, D), lambda r: (0, r, 0)),
            pl.BlockSpec((NC, R, 1), lambda r: (0, r, 0)),
            pl.BlockSpec((R, D), lambda r: (r, 0)),
            pl.BlockSpec((D, D), lambda r: (0, 0)),
            pl.BlockSpec((D, D), lambda r: (0, 0)),
            pl.BlockSpec((1, D), lambda r: (0, 0)),
        ],
        out_specs=pl.BlockSpec((R, D), lambda r: (r, 0)),
        out_shape=jax.ShapeDtypeStruct((N, D), jnp.float32),
    )


_TC_RELU = _make_tc(True)
_TC_LIN = _make_tc(False)


def kernel(x, edge_index, W1_l, W1_r, b1, W2_l, W2_r, b2):
    pad = EP - E
    src_p = jnp.concatenate(
        [edge_index[0], jnp.zeros((pad,), jnp.int32)]).reshape(TCH, CH)
    dst_pad = N + (jnp.arange(pad, dtype=jnp.int32) % (NP - N))
    dst_p = jnp.concatenate([edge_index[1], dst_pad]).reshape(TCH, CH)

    agg1, cnt1 = _make_sc(True)(x, src_p, dst_p)
    cnt3 = cnt1.reshape(NC, NP, 1)
    h = _TC_RELU(agg1, cnt3, x, W1_l, W1_r, b1.reshape(1, D))
    agg2, = _make_sc(False)(h, src_p, dst_p)
    return _TC_LIN(agg2, cnt3, h, W2_l, W2_r, b2.reshape(1, D))


# speedup vs baseline: 1.3419x; 1.1195x over previous
import functools

import jax
import jax.numpy as jnp
from jax import lax
from jax.experimental import pallas as pl
from jax.experimental.pallas import tpu as pltpu
from jax.experimental.pallas import tpu_sc as plsc

N = 10000
D = 128
E = 320000
NC = 2
NS = 16
CH = 128
TCH = 2560
CPF = 144
CPS = 16
PC = 16
PAIRS = PC // 2
EP = TCH * CH
NP = 10240
SPT = NP // NS
R = 1000


def _sc_body(with_cnt, *refs):
    if with_cnt:
        (x_hbm, src_hbm, dst_hbm, agg_out, cnt_out,
         agg_sh, sidx0, sidx1, didx0, didx1, rows0, rows1,
         gsem0, gsem1, isem, cnt_sh, ones_v) = refs
    else:
        (x_hbm, src_hbm, dst_hbm, agg_out,
         agg_sh, sidx0, sidx1, didx0, didx1, rows0, rows1,
         gsem0, gsem1, isem) = refs
    c = lax.axis_index("c")
    s = lax.axis_index("s")
    row0 = s * SPT

    zv = jnp.zeros((16,), jnp.float32)

    def _zb(i, carry):
        rows0[i // 8, pl.ds((i % 8) * 16, 16)] = zv
        return carry

    lax.fori_loop(0, CH * 8, _zb, 0)
    for k in range(SPT // CH):
        pltpu.sync_copy(rows0, agg_sh.at[pl.ds(row0 + k * CH, CH), :])
    if with_cnt:
        ov = jnp.ones((16,), jnp.float32)
        for k in range(CH // 16):
            ones_v[pl.ds(k * 16, 16)] = ov
        for k in range(SPT // CH):
            pltpu.sync_copy(rows0.at[0], cnt_sh.at[pl.ds(row0 + k * CH, CH)])

    def _pipeline(qbase, npieces):
        pltpu.sync_copy(src_hbm.at[pl.ds(qbase, PC), :], sidx0)
        pltpu.sync_copy(dst_hbm.at[pl.ds(qbase, PC), :], didx0)
        for p in range(npieces):
            sib, dib = (sidx0, didx0) if p % 2 == 0 else (sidx1, didx1)
            if p < npieces - 1:
                sib_n, dib_n = (sidx1, didx1) if p % 2 == 0 else (sidx0, didx0)
                ip = pltpu.async_copy(
                    src_hbm.at[pl.ds(qbase + (p + 1) * PC, PC), :], sib_n,
                    isem)
                ip2 = pltpu.async_copy(
                    dst_hbm.at[pl.ds(qbase + (p + 1) * PC, PC), :], dib_n,
                    isem)
            pltpu.async_copy(x_hbm.at[sib.at[0]], rows0, gsem0)

            def _pair(i, carry):
                j0 = 2 * i
                j1 = j0 + 1
                pltpu.async_copy(x_hbm.at[sib.at[j1]], rows1, gsem1)
                pltpu.make_async_copy(
                    x_hbm.at[sib.at[j0]], rows0, gsem0).wait()
                pltpu.sync_copy(rows0, agg_sh.at[dib.at[j0]], add=True)
                if with_cnt:
                    pltpu.sync_copy(ones_v, cnt_sh.at[dib.at[j0]], add=True)

                @pl.when(i < PAIRS - 1)
                def _():
                    pltpu.async_copy(x_hbm.at[sib.at[j0 + 2]], rows0, gsem0)

                pltpu.make_async_copy(
                    x_hbm.at[sib.at[j1]], rows1, gsem1).wait()
                pltpu.sync_copy(rows1, agg_sh.at[dib.at[j1]], add=True)
                if with_cnt:
                    pltpu.sync_copy(ones_v, cnt_sh.at[dib.at[j1]], add=True)
                return carry

            lax.fori_loop(0, PAIRS, _pair, 0)
            if p < npieces - 1:
                ip.wait()
                ip2.wait()

    plsc.subcore_barrier()

    @pl.when(c == 0)
    def _():
        _pipeline(s * CPF, CPF // PC)

    if CPS > 0:
        @pl.when(c == 1)
        def _():
            _pipeline(NS * CPF + s * CPS, CPS // PC)

    plsc.subcore_barrier()

    for k in range(SPT // CH):
        pltpu.sync_copy(agg_sh.at[pl.ds(row0 + k * CH, CH), :],
                        agg_out.at[c, pl.ds(row0 + k * CH, CH), :])
    if with_cnt:
        pltpu.sync_copy(cnt_sh.at[pl.ds(row0, SPT)],
                        cnt_out.at[c, pl.ds(row0, SPT)])


@functools.cache
def _make_sc(with_cnt):
    mesh = plsc.VectorSubcoreMesh(core_axis_name="c", subcore_axis_name="s",
                                  num_cores=NC, num_subcores=NS)
    out_type = [jax.ShapeDtypeStruct((NC, NP, D), jnp.float32)]
    scratch = [
        pltpu.VMEM_SHARED((NP, D), jnp.float32),
        pltpu.VMEM((PC, CH), jnp.int32),
        pltpu.VMEM((PC, CH), jnp.int32),
        pltpu.VMEM((PC, CH), jnp.int32),
        pltpu.VMEM((PC, CH), jnp.int32),
        pltpu.VMEM((CH, D), jnp.float32),
        pltpu.VMEM((CH, D), jnp.float32),
        pltpu.SemaphoreType.DMA,
        pltpu.SemaphoreType.DMA,
        pltpu.SemaphoreType.DMA,
    ]
    if with_cnt:
        out_type.append(jax.ShapeDtypeStruct((NC, NP), jnp.float32))
        scratch += [
            pltpu.VMEM_SHARED((NP,), jnp.float32),
            pltpu.VMEM((CH,), jnp.float32),
        ]
    return pl.kernel(
        functools.partial(_sc_body, with_cnt),
        out_type=out_type,
        mesh=mesh,
        scratch_types=scratch,
    )


def _tc_body(relu, agg_ref, cnt_ref, xin_ref, wl_ref, wr_ref, b_ref, out_ref):
    cnt = cnt_ref[0] + cnt_ref[1]
    rec = 1.0 / jnp.maximum(cnt, 1.0)
    mean = (agg_ref[0] + agg_ref[1]) * rec
    acc = jnp.dot(mean, wl_ref[...], preferred_element_type=jnp.float32)
    acc = acc + jnp.dot(xin_ref[...], wr_ref[...],
                        preferred_element_type=jnp.float32)
    acc = acc + b_ref[...]
    out_ref[...] = jnp.maximum(acc, 0.0) if relu else acc


def _make_tc(relu):
    return pl.pallas_call(
        functools.partial(_tc_body, relu),
        grid=(N // R,),
        in_specs=[
            pl.BlockSpec((NC, R, D), lambda r: (0, r, 0)),
            pl.BlockSpec((NC, R, 1), lambda r: (0, r, 0)),
            pl.BlockSpec((R, D), lambda r: (r, 0)),
            pl.BlockSpec((D, D), lambda r: (0, 0)),
            pl.BlockSpec((D, D), lambda r: (0, 0)),
            pl.BlockSpec((1, D), lambda r: (0, 0)),
        ],
        out_specs=pl.BlockSpec((R, D), lambda r: (r, 0)),
        out_shape=jax.ShapeDtypeStruct((N, D), jnp.float32),
    )


_TC_RELU = _make_tc(True)
_TC_LIN = _make_tc(False)


def kernel(x, edge_index, W1_l, W1_r, b1, W2_l, W2_r, b2):
    pad = EP - E
    src_p = jnp.concatenate(
        [edge_index[0], jnp.zeros((pad,), jnp.int32)]).reshape(TCH, CH)
    dst_pad = N + (jnp.arange(pad, dtype=jnp.int32) % (NP - N))
    dst_p = jnp.concatenate([edge_index[1], dst_pad]).reshape(TCH, CH)

    agg1, cnt1 = _make_sc(True)(x, src_p, dst_p)
    cnt3 = cnt1.reshape(NC, NP, 1)
    h = _TC_RELU(agg1, cnt3, x, W1_l, W1_r, b1.reshape(1, D))
    agg2, = _make_sc(False)(h, src_p, dst_p)
    return _TC_LIN(agg2, cnt3, h, W2_l, W2_r, b2.reshape(1, D))


# speedup vs baseline: 1.3504x; 1.0064x over previous
import functools

import jax
import jax.numpy as jnp
from jax import lax
from jax.experimental import pallas as pl
from jax.experimental.pallas import tpu as pltpu
from jax.experimental.pallas import tpu_sc as plsc

N = 10000
D = 128
E = 320000
NC = 2
NS = 16
CH = 128
TCH = 2560
CPF = 152
CPS = 8
PC = 8
PAIRS = PC // 2
EP = TCH * CH
NP = 10240
SPT = NP // NS
R = 1000


def _sc_body(with_cnt, *refs):
    if with_cnt:
        (x_hbm, src_hbm, dst_hbm, agg_out, cnt_out,
         agg_sh, sidx0, sidx1, didx0, didx1, rows0, rows1,
         gsem0, gsem1, isem, cnt_sh, ones_v) = refs
    else:
        (x_hbm, src_hbm, dst_hbm, agg_out,
         agg_sh, sidx0, sidx1, didx0, didx1, rows0, rows1,
         gsem0, gsem1, isem) = refs
    c = lax.axis_index("c")
    s = lax.axis_index("s")
    row0 = s * SPT

    zv = jnp.zeros((16,), jnp.float32)

    def _zb(i, carry):
        rows0[i // 8, pl.ds((i % 8) * 16, 16)] = zv
        return carry

    lax.fori_loop(0, CH * 8, _zb, 0)
    for k in range(SPT // CH):
        pltpu.sync_copy(rows0, agg_sh.at[pl.ds(row0 + k * CH, CH), :])
    if with_cnt:
        ov = jnp.ones((16,), jnp.float32)
        for k in range(CH // 16):
            ones_v[pl.ds(k * 16, 16)] = ov
        for k in range(SPT // CH):
            pltpu.sync_copy(rows0.at[0], cnt_sh.at[pl.ds(row0 + k * CH, CH)])

    def _pipeline(qbase, npieces):
        pltpu.sync_copy(src_hbm.at[pl.ds(qbase, PC), :], sidx0)
        pltpu.sync_copy(dst_hbm.at[pl.ds(qbase, PC), :], didx0)
        for p in range(npieces):
            sib, dib = (sidx0, didx0) if p % 2 == 0 else (sidx1, didx1)
            if p < npieces - 1:
                sib_n, dib_n = (sidx1, didx1) if p % 2 == 0 else (sidx0, didx0)
                ip = pltpu.async_copy(
                    src_hbm.at[pl.ds(qbase + (p + 1) * PC, PC), :], sib_n,
                    isem)
                ip2 = pltpu.async_copy(
                    dst_hbm.at[pl.ds(qbase + (p + 1) * PC, PC), :], dib_n,
                    isem)
            pltpu.async_copy(x_hbm.at[sib.at[0]], rows0, gsem0)

            def _pair(i, carry):
                j0 = 2 * i
                j1 = j0 + 1
                pltpu.async_copy(x_hbm.at[sib.at[j1]], rows1, gsem1)
                pltpu.make_async_copy(
                    x_hbm.at[sib.at[j0]], rows0, gsem0).wait()
                pltpu.sync_copy(rows0, agg_sh.at[dib.at[j0]], add=True)
                if with_cnt:
                    pltpu.sync_copy(ones_v, cnt_sh.at[dib.at[j0]], add=True)

                @pl.when(i < PAIRS - 1)
                def _():
                    pltpu.async_copy(x_hbm.at[sib.at[j0 + 2]], rows0, gsem0)

                pltpu.make_async_copy(
                    x_hbm.at[sib.at[j1]], rows1, gsem1).wait()
                pltpu.sync_copy(rows1, agg_sh.at[dib.at[j1]], add=True)
                if with_cnt:
                    pltpu.sync_copy(ones_v, cnt_sh.at[dib.at[j1]], add=True)
                return carry

            lax.fori_loop(0, PAIRS, _pair, 0)
            if p < npieces - 1:
                ip.wait()
                ip2.wait()

    plsc.subcore_barrier()

    @pl.when(c == 0)
    def _():
        _pipeline(s * CPF, CPF // PC)

    if CPS > 0:
        @pl.when(c == 1)
        def _():
            _pipeline(NS * CPF + s * CPS, CPS // PC)

    plsc.subcore_barrier()

    for k in range(SPT // CH):
        pltpu.sync_copy(agg_sh.at[pl.ds(row0 + k * CH, CH), :],
                        agg_out.at[c, pl.ds(row0 + k * CH, CH), :])
    if with_cnt:
        pltpu.sync_copy(cnt_sh.at[pl.ds(row0, SPT)],
                        cnt_out.at[c, pl.ds(row0, SPT)])


@functools.cache
def _make_sc(with_cnt):
    mesh = plsc.VectorSubcoreMesh(core_axis_name="c", subcore_axis_name="s",
                                  num_cores=NC, num_subcores=NS)
    out_type = [jax.ShapeDtypeStruct((NC, NP, D), jnp.float32)]
    scratch = [
        pltpu.VMEM_SHARED((NP, D), jnp.float32),
        pltpu.VMEM((PC, CH), jnp.int32),
        pltpu.VMEM((PC, CH), jnp.int32),
        pltpu.VMEM((PC, CH), jnp.int32),
        pltpu.VMEM((PC, CH), jnp.int32),
        pltpu.VMEM((CH, D), jnp.float32),
        pltpu.VMEM((CH, D), jnp.float32),
        pltpu.SemaphoreType.DMA,
        pltpu.SemaphoreType.DMA,
        pltpu.SemaphoreType.DMA,
    ]
    if with_cnt:
        out_type.append(jax.ShapeDtypeStruct((NC, NP), jnp.float32))
        scratch += [
            pltpu.VMEM_SHARED((NP,), jnp.float32),
            pltpu.VMEM((CH,), jnp.float32),
        ]
    return pl.kernel(
        functools.partial(_sc_body, with_cnt),
        out_type=out_type,
        mesh=mesh,
        scratch_types=scratch,
    )


def _tc_body(relu, agg_ref, cnt_ref, xin_ref, wl_ref, wr_ref, b_ref, out_ref):
    cnt = cnt_ref[0] + cnt_ref[1]
    rec = 1.0 / jnp.maximum(cnt, 1.0)
    mean = (agg_ref[0] + agg_ref[1]) * rec
    acc = jnp.dot(mean, wl_ref[...], preferred_element_type=jnp.float32)
    acc = acc + jnp.dot(xin_ref[...], wr_ref[...],
                        preferred_element_type=jnp.float32)
    acc = acc + b_ref[...]
    out_ref[...] = jnp.maximum(acc, 0.0) if relu else acc


def _make_tc(relu):
    return pl.pallas_call(
        functools.partial(_tc_body, relu),
        grid=(N // R,),
        in_specs=[
            pl.BlockSpec((NC, R, D), lambda r: (0, r, 0)),
            pl.BlockSpec((NC, R, 1), lambda r: (0, r, 0)),
            pl.BlockSpec((R, D), lambda r: (r, 0)),
            pl.BlockSpec((D, D), lambda r: (0, 0)),
            pl.BlockSpec((D, D), lambda r: (0, 0)),
            pl.BlockSpec((1, D), lambda r: (0, 0)),
        ],
        out_specs=pl.BlockSpec((R, D), lambda r: (r, 0)),
        out_shape=jax.ShapeDtypeStruct((N, D), jnp.float32),
    )


_TC_RELU = _make_tc(True)
_TC_LIN = _make_tc(False)


def kernel(x, edge_index, W1_l, W1_r, b1, W2_l, W2_r, b2):
    pad = EP - E
    src_p = jnp.concatenate(
        [edge_index[0], jnp.zeros((pad,), jnp.int32)]).reshape(TCH, CH)
    dst_pad = N + (jnp.arange(pad, dtype=jnp.int32) % (NP - N))
    dst_p = jnp.concatenate([edge_index[1], dst_pad]).reshape(TCH, CH)

    agg1, cnt1 = _make_sc(True)(x, src_p, dst_p)
    cnt3 = cnt1.reshape(NC, NP, 1)
    h = _TC_RELU(agg1, cnt3, x, W1_l, W1_r, b1.reshape(1, D))
    agg2, = _make_sc(False)(h, src_p, dst_p)
    return _TC_LIN(agg2, cnt3, h, W2_l, W2_r, b2.reshape(1, D))


# speedup vs baseline: 3.7552x; 2.7807x over previous
import functools

import jax
import jax.numpy as jnp
from jax import lax
from jax.experimental import pallas as pl
from jax.experimental.pallas import tpu as pltpu
from jax.experimental.pallas import tpu_sc as plsc

N = 10000
D = 128
E = 320000
NC = 2
NS = 16
CH = 128
TCH = 2560
CPF = 80
CPS = 80
PC = 16
PAIRS = PC // 2
EP = TCH * CH
NP = 10240
SPT = NP // NS
R = 1000


def _sc_body(with_cnt, *refs):
    if with_cnt:
        (x_hbm, src_hbm, dst_hbm, agg_out, cnt_out,
         agg_sh, sidx0, sidx1, didx0, didx1, rows0, rows1,
         gsem0, gsem1, isem, cnt_sh, ones_v) = refs
    else:
        (x_hbm, src_hbm, dst_hbm, agg_out,
         agg_sh, sidx0, sidx1, didx0, didx1, rows0, rows1,
         gsem0, gsem1, isem) = refs
    c = lax.axis_index("c")
    s = lax.axis_index("s")
    row0 = s * SPT

    zv = jnp.zeros((16,), jnp.float32)

    def _zb(i, carry):
        rows0[i // 8, pl.ds((i % 8) * 16, 16)] = zv
        return carry

    lax.fori_loop(0, CH * 8, _zb, 0)
    for k in range(SPT // CH):
        pltpu.sync_copy(rows0, agg_sh.at[pl.ds(row0 + k * CH, CH), :])
    if with_cnt:
        ov = jnp.ones((16,), jnp.float32)
        for k in range(CH // 16):
            ones_v[pl.ds(k * 16, 16)] = ov
        for k in range(SPT // CH):
            pltpu.sync_copy(rows0.at[0], cnt_sh.at[pl.ds(row0 + k * CH, CH)])

    def _pipeline(qbase, npieces):
        pltpu.sync_copy(src_hbm.at[pl.ds(qbase, PC), :], sidx0)
        pltpu.sync_copy(dst_hbm.at[pl.ds(qbase, PC), :], didx0)
        for p in range(npieces):
            sib, dib = (sidx0, didx0) if p % 2 == 0 else (sidx1, didx1)
            if p < npieces - 1:
                sib_n, dib_n = (sidx1, didx1) if p % 2 == 0 else (sidx0, didx0)
                ip = pltpu.async_copy(
                    src_hbm.at[pl.ds(qbase + (p + 1) * PC, PC), :], sib_n,
                    isem)
                ip2 = pltpu.async_copy(
                    dst_hbm.at[pl.ds(qbase + (p + 1) * PC, PC), :], dib_n,
                    isem)
            pltpu.async_copy(x_hbm.at[sib.at[0]], rows0, gsem0)

            def _pair(i, carry):
                j0 = 2 * i
                j1 = j0 + 1
                pltpu.async_copy(x_hbm.at[sib.at[j1]], rows1, gsem1)
                pltpu.make_async_copy(
                    x_hbm.at[sib.at[j0]], rows0, gsem0).wait()
                pltpu.sync_copy(rows0, agg_sh.at[dib.at[j0]], add=True)
                if with_cnt:
                    pltpu.sync_copy(ones_v, cnt_sh.at[dib.at[j0]], add=True)

                @pl.when(i < PAIRS - 1)
                def _():
                    pltpu.async_copy(x_hbm.at[sib.at[j0 + 2]], rows0, gsem0)

                pltpu.make_async_copy(
                    x_hbm.at[sib.at[j1]], rows1, gsem1).wait()
                pltpu.sync_copy(rows1, agg_sh.at[dib.at[j1]], add=True)
                if with_cnt:
                    pltpu.sync_copy(ones_v, cnt_sh.at[dib.at[j1]], add=True)
                return carry

            lax.fori_loop(0, PAIRS, _pair, 0)
            if p < npieces - 1:
                ip.wait()
                ip2.wait()

    plsc.subcore_barrier()

    @pl.when(c == 0)
    def _():
        _pipeline(s * CPF, CPF // PC)

    if CPS > 0:
        @pl.when(c == 1)
        def _():
            _pipeline(NS * CPF + s * CPS, CPS // PC)

    plsc.subcore_barrier()

    for k in range(SPT // CH):
        pltpu.sync_copy(agg_sh.at[pl.ds(row0 + k * CH, CH), :],
                        agg_out.at[c, pl.ds(row0 + k * CH, CH), :])
    if with_cnt:
        pltpu.sync_copy(cnt_sh.at[pl.ds(row0, SPT)],
                        cnt_out.at[c, pl.ds(row0, SPT)])


@functools.cache
def _make_sc(with_cnt):
    mesh = plsc.VectorSubcoreMesh(core_axis_name="c", subcore_axis_name="s",
                                  num_cores=NC, num_subcores=NS)
    out_type = [jax.ShapeDtypeStruct((NC, NP, D), jnp.float32)]
    scratch = [
        pltpu.VMEM_SHARED((NP, D), jnp.float32),
        pltpu.VMEM((PC, CH), jnp.int32),
        pltpu.VMEM((PC, CH), jnp.int32),
        pltpu.VMEM((PC, CH), jnp.int32),
        pltpu.VMEM((PC, CH), jnp.int32),
        pltpu.VMEM((CH, D), jnp.float32),
        pltpu.VMEM((CH, D), jnp.float32),
        pltpu.SemaphoreType.DMA,
        pltpu.SemaphoreType.DMA,
        pltpu.SemaphoreType.DMA,
    ]
    if with_cnt:
        out_type.append(jax.ShapeDtypeStruct((NC, NP), jnp.float32))
        scratch += [
            pltpu.VMEM_SHARED((NP,), jnp.float32),
            pltpu.VMEM((CH,), jnp.float32),
        ]
    return pl.kernel(
        functools.partial(_sc_body, with_cnt),
        out_type=out_type,
        mesh=mesh,
        scratch_types=scratch,
    )


def _tc_body(relu, agg_ref, cnt_ref, xin_ref, wl_ref, wr_ref, b_ref, out_ref):
    cnt = cnt_ref[0] + cnt_ref[1]
    rec = 1.0 / jnp.maximum(cnt, 1.0)
    mean = (agg_ref[0] + agg_ref[1]) * rec
    acc = jnp.dot(mean, wl_ref[...], preferred_element_type=jnp.float32)
    acc = acc + jnp.dot(xin_ref[...], wr_ref[...],
                        preferred_element_type=jnp.float32)
    acc = acc + b_ref[...]
    out_ref[...] = jnp.maximum(acc, 0.0) if relu else acc


def _make_tc(relu):
    return pl.pallas_call(
        functools.partial(_tc_body, relu),
        grid=(N // R,),
        in_specs=[
            pl.BlockSpec((NC, R, D), lambda r: (0, r, 0)),
            pl.BlockSpec((NC, R, 1), lambda r: (0, r, 0)),
            pl.BlockSpec((R, D), lambda r: (r, 0)),
            pl.BlockSpec((D, D), lambda r: (0, 0)),
            pl.BlockSpec((D, D), lambda r: (0, 0)),
            pl.BlockSpec((1, D), lambda r: (0, 0)),
        ],
        out_specs=pl.BlockSpec((R, D), lambda r: (r, 0)),
        out_shape=jax.ShapeDtypeStruct((N, D), jnp.float32),
    )


_TC_RELU = _make_tc(True)
_TC_LIN = _make_tc(False)


def kernel(x, edge_index, W1_l, W1_r, b1, W2_l, W2_r, b2):
    pad = EP - E
    src_pad = (jnp.arange(pad, dtype=jnp.int32) * 131) % N
    src_p = jnp.concatenate([edge_index[0], src_pad]).reshape(TCH, CH)
    dst_pad = N + (jnp.arange(pad, dtype=jnp.int32) % (NP - N))
    dst_p = jnp.concatenate([edge_index[1], dst_pad]).reshape(TCH, CH)

    agg1, cnt1 = _make_sc(True)(x, src_p, dst_p)
    cnt3 = cnt1.reshape(NC, NP, 1)
    h = _TC_RELU(agg1, cnt3, x, W1_l, W1_r, b1.reshape(1, D))
    agg2, = _make_sc(False)(h, src_p, dst_p)
    return _TC_LIN(agg2, cnt3, h, W2_l, W2_r, b2.reshape(1, D))
